# Initial kernel scaffold; baseline (speedup 1.0000x reference)
import jax
import jax.numpy as jnp
from jax.experimental import pallas as pl


def kernel(x, edge_index, edge_attr, W1, b1, W2, b2):
    raise NotImplementedError("write your pallas kernel here")


# speedup vs baseline: 4.7293x; 4.7293x over previous
import functools

import jax
import jax.numpy as jnp
from jax import lax
from jax.experimental import pallas as pl
from jax.experimental.pallas import tpu as pltpu
from jax.experimental.pallas import tpu_sc as plsc

N = 10000
E = 320000
D = 128
DE = 16
NPAD = 10240
NC, NS = 2, 16
NW = NC * NS
EPW = E // NW
C = 80
NCH = EPW // C
HN = NPAD // 2
ACCH = 6144
RPT = ACCH // NS
ZR = RPT // 4

_mesh = plsc.VectorSubcoreMesh(core_axis_name="c", subcore_axis_name="s")


@functools.partial(
    pl.kernel,
    out_type=(
        jax.ShapeDtypeStruct((NC, 2, ACCH, DE), jnp.float32),
        jax.ShapeDtypeStruct((NC, 2, ACCH, DE), jnp.float32),
    ),
    mesh=_mesh,
    scratch_types=[
        pltpu.VMEM((C,), jnp.int32),
        pltpu.VMEM((C,), jnp.int32),
        pltpu.VMEM((C, DE), jnp.float32),
        pltpu.VMEM((C, DE), jnp.float32),
        pltpu.VMEM((RPT, DE), jnp.float32),
        pltpu.VMEM_SHARED((ACCH, DE), jnp.float32),
        pltpu.VMEM_SHARED((ACCH, DE), jnp.float32),
    ],
)
def _edge_agg_deg(rowp, colp, attr, e0, z16, agg_out, deg_out,
                  ridx, cidx, attr_v, e0_v, z_v, agg_s, deg_s):
    c = lax.axis_index("c")
    s = lax.axis_index("s")
    wid = c * NS + s
    pltpu.sync_copy(e0, e0_v)
    pltpu.sync_copy(z16, z_v)

    for p in range(2):
        pltpu.sync_copy(z_v, agg_s.at[pl.ds(s * RPT, RPT)])
        pltpu.sync_copy(z_v, deg_s.at[pl.ds(s * RPT, RPT)])
        plsc.subcore_barrier()

        def body(j, carry):
            pltpu.sync_copy(rowp.at[p, wid, j], ridx)
            pltpu.sync_copy(colp.at[p, wid, j], cidx)
            pltpu.sync_copy(attr.at[pl.ds(wid * EPW + j * C, C)], attr_v)
            pltpu.sync_copy(attr_v, agg_s.at[ridx], add=True)
            pltpu.sync_copy(e0_v, deg_s.at[cidx], add=True)
            return carry

        lax.fori_loop(0, NCH, body, 0)
        plsc.subcore_barrier()
        pltpu.sync_copy(agg_s.at[pl.ds(s * RPT, RPT)],
                        agg_out.at[c, p, pl.ds(s * RPT, RPT)])
        pltpu.sync_copy(deg_s.at[pl.ds(s * RPT, RPT)],
                        deg_out.at[c, p, pl.ds(s * RPT, RPT)])


@functools.partial(
    pl.kernel,
    out_type=jax.ShapeDtypeStruct((NC, 2, ACCH, D), jnp.float32),
    mesh=_mesh,
    scratch_types=[
        pltpu.VMEM((C,), jnp.int32),
        pltpu.VMEM((C,), jnp.int32),
        pltpu.VMEM((C, D), jnp.float32),
        pltpu.VMEM((ZR, D), jnp.float32),
        pltpu.VMEM_SHARED((ACCH, D), jnp.float32),
        pltpu.SemaphoreType.DMA,
    ],
)
def _spmm(hs, row3, colp, z128, out, ridx, cidx, gbuf, zbuf, acc_s, sem):
    c = lax.axis_index("c")
    s = lax.axis_index("s")
    wid = c * NS + s
    pltpu.sync_copy(z128, zbuf)
    for p in range(2):
        for k in range(4):
            pltpu.sync_copy(zbuf, acc_s.at[pl.ds(s * RPT + k * ZR, ZR)])
        plsc.subcore_barrier()

        def body(j, carry):
            pltpu.sync_copy(row3.at[wid, j], ridx)
            pltpu.sync_copy(colp.at[p, wid, j], cidx)
            pltpu.async_copy(hs.at[ridx], gbuf, sem).wait()
            pltpu.sync_copy(gbuf, acc_s.at[cidx], add=True)
            return carry

        lax.fori_loop(0, NCH, body, 0)
        plsc.subcore_barrier()
        pltpu.sync_copy(acc_s.at[pl.ds(s * RPT, RPT)],
                        out.at[c, p, pl.ds(s * RPT, RPT)])


_BLK = 1024
_GRID = NPAD // _BLK
_HB = HN // _BLK


def _dense1_body(x_ref, aggp_ref, degp_ref, w1x_ref, w1a_ref,
                 hs_ref, dinv_ref, agg_ref):
    agg = aggp_ref[0, 0] + aggp_ref[1, 0]
    deg = degp_ref[0, 0, :, 0:1] + degp_ref[1, 0, :, 0:1] + 1.0
    dinv = lax.rsqrt(deg)
    hl = (jnp.dot(x_ref[...], w1x_ref[...], preferred_element_type=jnp.float32)
          + jnp.dot(agg, w1a_ref[...], preferred_element_type=jnp.float32))
    hs_ref[...] = dinv * hl
    dinv_ref[...] = dinv
    agg_ref[...] = agg


def _dense2_body(accp_ref, hs1_ref, dinv_ref, agg_ref, w2x_ref, w2a_ref,
                 b1_ref, hs2_ref):
    dinv = dinv_ref[...]
    pre = accp_ref[0, 0] + accp_ref[1, 0] + hs1_ref[...]
    h1 = jnp.maximum(dinv * pre + b1_ref[...], 0.0)
    h2l = (jnp.dot(h1, w2x_ref[...], preferred_element_type=jnp.float32)
           + jnp.dot(agg_ref[...], w2a_ref[...],
                     preferred_element_type=jnp.float32))
    hs2_ref[...] = dinv * h2l


def _finish_body(accp_ref, hs2_ref, dinv_ref, b2_ref, out_ref):
    pre = accp_ref[0, 0] + accp_ref[1, 0] + hs2_ref[...]
    out_ref[...] = dinv_ref[...] * pre + b2_ref[...]


def _rows(i):
    return (i, 0)


def _full(i):
    return (0, 0)


def _acc_map(i):
    return (0, i // _HB, i % _HB, 0)


_spec_nd = pl.BlockSpec((_BLK, D), _rows)
_spec_nde = pl.BlockSpec((_BLK, DE), _rows)
_spec_n1 = pl.BlockSpec((_BLK, 1), _rows)
_spec_aggp = pl.BlockSpec((NC, 1, _BLK, DE), _acc_map)
_spec_accp = pl.BlockSpec((NC, 1, _BLK, D), _acc_map)
_spec_wx = pl.BlockSpec((D, D), _full)
_spec_wa = pl.BlockSpec((DE, D), _full)
_spec_b = pl.BlockSpec((1, D), _full)

_dense1 = pl.pallas_call(
    _dense1_body,
    grid=(_GRID,),
    in_specs=[_spec_nd, _spec_aggp, _spec_aggp, _spec_wx, _spec_wa],
    out_specs=[_spec_nd, _spec_n1, _spec_nde],
    out_shape=[
        jax.ShapeDtypeStruct((NPAD, D), jnp.float32),
        jax.ShapeDtypeStruct((NPAD, 1), jnp.float32),
        jax.ShapeDtypeStruct((NPAD, DE), jnp.float32),
    ],
)

_dense2 = pl.pallas_call(
    _dense2_body,
    grid=(_GRID,),
    in_specs=[_spec_accp, _spec_nd, _spec_n1, _spec_nde, _spec_wx,
              _spec_wa, _spec_b],
    out_specs=_spec_nd,
    out_shape=jax.ShapeDtypeStruct((NPAD, D), jnp.float32),
)

_finish = pl.pallas_call(
    _finish_body,
    grid=(_GRID,),
    in_specs=[_spec_accp, _spec_nd, _spec_n1, _spec_b],
    out_specs=_spec_nd,
    out_shape=jax.ShapeDtypeStruct((NPAD, D), jnp.float32),
)


def kernel(x, edge_index, edge_attr, W1, b1, W2, b2):
    row = edge_index[0]
    col = edge_index[1]
    dummy = HN + (jnp.arange(E, dtype=jnp.int32) % 128)
    rowp = jnp.stack([
        jnp.where(row < HN, row, dummy),
        jnp.where(row >= HN, row - HN, dummy),
    ]).reshape(2, NW, NCH, C)
    colp = jnp.stack([
        jnp.where(col < HN, col, dummy),
        jnp.where(col >= HN, col - HN, dummy),
    ]).reshape(2, NW, NCH, C)
    row3 = row.reshape(NW, NCH, C)
    x_pad = jnp.pad(x, ((0, NPAD - N), (0, 0)))
    e0 = jnp.concatenate(
        [jnp.ones((C, 1), jnp.float32), jnp.zeros((C, DE - 1), jnp.float32)],
        axis=1)
    z16 = jnp.zeros((RPT, DE), jnp.float32)
    z128 = jnp.zeros((ZR, D), jnp.float32)
    W1x, W1a = W1[:D], W1[D:]
    W2x, W2a = W2[:D], W2[D:]

    agg_p, deg_p = _edge_agg_deg(rowp, colp, edge_attr, e0, z16)
    hs1, dinv, agg = _dense1(x_pad, agg_p, deg_p, W1x, W1a)
    acc1 = _spmm(hs1, row3, colp, z128)
    hs2 = _dense2(acc1, hs1, dinv, agg, W2x, W2a, b1.reshape(1, D))
    acc2 = _spmm(hs2, row3, colp, z128)
    out = _finish(acc2, hs2, dinv, b2.reshape(1, D))
    return out[:N]


# speedup vs baseline: 6.3170x; 1.3357x over previous
import functools

import jax
import jax.numpy as jnp
from jax import lax
from jax.experimental import pallas as pl
from jax.experimental.pallas import tpu as pltpu
from jax.experimental.pallas import tpu_sc as plsc

N = 10000
E = 320000
D = 128
DE = 16
NPAD = 10240
NC, NS = 2, 16
NW = NC * NS
EPW = E // NW
C = 80
NCH = EPW // C
HN = NPAD // 2
ACCH = 6144
RPT = ACCH // NS
ZR = RPT // 4

_mesh = plsc.VectorSubcoreMesh(core_axis_name="c", subcore_axis_name="s")


@functools.partial(
    pl.kernel,
    out_type=(
        jax.ShapeDtypeStruct((NC, 2, ACCH, DE), jnp.float32),
        jax.ShapeDtypeStruct((NC, 2, ACCH, DE), jnp.float32),
    ),
    mesh=_mesh,
    scratch_types=[
        pltpu.VMEM((C,), jnp.int32),
        pltpu.VMEM((C,), jnp.int32),
        pltpu.VMEM((C, DE), jnp.float32),
        pltpu.VMEM((C, DE), jnp.float32),
        pltpu.VMEM((RPT, DE), jnp.float32),
        pltpu.VMEM_SHARED((ACCH, DE), jnp.float32),
        pltpu.VMEM_SHARED((ACCH, DE), jnp.float32),
    ],
)
def _edge_agg_deg(rowp, colp, attr, e0, z16, agg_out, deg_out,
                  ridx, cidx, attr_v, e0_v, z_v, agg_s, deg_s):
    c = lax.axis_index("c")
    s = lax.axis_index("s")
    wid = c * NS + s
    pltpu.sync_copy(e0, e0_v)
    pltpu.sync_copy(z16, z_v)

    for p in range(2):
        pltpu.sync_copy(z_v, agg_s.at[pl.ds(s * RPT, RPT)])
        pltpu.sync_copy(z_v, deg_s.at[pl.ds(s * RPT, RPT)])
        plsc.subcore_barrier()

        def body(j, carry):
            pltpu.sync_copy(rowp.at[p, wid, j], ridx)
            pltpu.sync_copy(colp.at[p, wid, j], cidx)
            pltpu.sync_copy(attr.at[pl.ds(wid * EPW + j * C, C)], attr_v)
            pltpu.sync_copy(attr_v, agg_s.at[ridx], add=True)
            pltpu.sync_copy(e0_v, deg_s.at[cidx], add=True)
            return carry

        lax.fori_loop(0, NCH, body, 0)
        plsc.subcore_barrier()
        pltpu.sync_copy(agg_s.at[pl.ds(s * RPT, RPT)],
                        agg_out.at[c, p, pl.ds(s * RPT, RPT)])
        pltpu.sync_copy(deg_s.at[pl.ds(s * RPT, RPT)],
                        deg_out.at[c, p, pl.ds(s * RPT, RPT)])


@functools.partial(
    pl.kernel,
    out_type=jax.ShapeDtypeStruct((NC, 2, ACCH, D), jnp.float32),
    mesh=_mesh,
    scratch_types=[
        pltpu.VMEM((C,), jnp.int32),
        pltpu.VMEM((C,), jnp.int32),
        pltpu.VMEM((C,), jnp.int32),
        pltpu.VMEM((C,), jnp.int32),
        pltpu.VMEM((C, D), jnp.float32),
        pltpu.VMEM((C, D), jnp.float32),
        pltpu.VMEM((ZR, D), jnp.float32),
        pltpu.VMEM_SHARED((ACCH, D), jnp.float32),
        pltpu.SemaphoreType.DMA,
        pltpu.SemaphoreType.DMA,
        pltpu.SemaphoreType.DMA,
        pltpu.SemaphoreType.DMA,
    ],
)
def _spmm(hs, row3, colp, z128, out, ridx0, ridx1, cidx0, cidx1,
          gbuf0, gbuf1, zbuf, acc_s, sg0, sg1, ss0, ss1):
    c = lax.axis_index("c")
    s = lax.axis_index("s")
    wid = c * NS + s
    ridx = (ridx0, ridx1)
    cidx = (cidx0, cidx1)
    gbuf = (gbuf0, gbuf1)
    sg = (sg0, sg1)
    ss = (ss0, ss1)
    pltpu.sync_copy(z128, zbuf)
    for p in range(2):
        for k in range(4):
            pltpu.sync_copy(zbuf, acc_s.at[pl.ds(s * RPT + k * ZR, ZR)])
        plsc.subcore_barrier()

        for b in range(2):
            pltpu.sync_copy(row3.at[wid, b], ridx[b])
            pltpu.sync_copy(colp.at[p, wid, b], cidx[b])
            pltpu.async_copy(hs.at[ridx[b]], gbuf[b], sg[b])

        def body(i, carry):
            for b in range(2):
                jj = 2 * i + b
                pltpu.make_async_copy(hs.at[ridx[b]], gbuf[b], sg[b]).wait()
                pltpu.async_copy(gbuf[b], acc_s.at[cidx[b]], ss[b], add=True)
                if b == 0:
                    nxt = jj + 2
                    pltpu.make_async_copy(
                        gbuf[b], acc_s.at[cidx[b]], ss[b]).wait()
                    pltpu.sync_copy(row3.at[wid, nxt], ridx[b])
                    pltpu.sync_copy(colp.at[p, wid, nxt], cidx[b])
                    pltpu.async_copy(hs.at[ridx[b]], gbuf[b], sg[b])
                else:
                    @pl.when(jj + 2 < NCH)
                    def _():
                        nxt = jj + 2
                        pltpu.make_async_copy(
                            gbuf[b], acc_s.at[cidx[b]], ss[b]).wait()
                        pltpu.sync_copy(row3.at[wid, nxt], ridx[b])
                        pltpu.sync_copy(colp.at[p, wid, nxt], cidx[b])
                        pltpu.async_copy(hs.at[ridx[b]], gbuf[b], sg[b])
            return carry

        lax.fori_loop(0, (NCH - 1) // 2, body, 0)
        pltpu.make_async_copy(hs.at[ridx[0]], gbuf[0], sg[0]).wait()
        pltpu.async_copy(gbuf[0], acc_s.at[cidx[0]], ss[0], add=True)
        pltpu.make_async_copy(gbuf[0], acc_s.at[cidx[0]], ss[0]).wait()
        pltpu.make_async_copy(gbuf[1], acc_s.at[cidx[1]], ss[1]).wait()
        plsc.subcore_barrier()
        pltpu.sync_copy(acc_s.at[pl.ds(s * RPT, RPT)],
                        out.at[c, p, pl.ds(s * RPT, RPT)])


_BLK = 1024
_GRID = NPAD // _BLK
_HB = HN // _BLK


def _dense1_body(x_ref, aggp_ref, degp_ref, w1x_ref, w1a_ref,
                 hs_ref, dinv_ref, agg_ref):
    agg = aggp_ref[0, 0] + aggp_ref[1, 0]
    deg = degp_ref[0, 0, :, 0:1] + degp_ref[1, 0, :, 0:1] + 1.0
    dinv = lax.rsqrt(deg)
    hl = (jnp.dot(x_ref[...], w1x_ref[...], preferred_element_type=jnp.float32)
          + jnp.dot(agg, w1a_ref[...], preferred_element_type=jnp.float32))
    hs_ref[...] = dinv * hl
    dinv_ref[...] = dinv
    agg_ref[...] = agg


def _dense2_body(accp_ref, hs1_ref, dinv_ref, agg_ref, w2x_ref, w2a_ref,
                 b1_ref, hs2_ref):
    dinv = dinv_ref[...]
    pre = accp_ref[0, 0] + accp_ref[1, 0] + hs1_ref[...]
    h1 = jnp.maximum(dinv * pre + b1_ref[...], 0.0)
    h2l = (jnp.dot(h1, w2x_ref[...], preferred_element_type=jnp.float32)
           + jnp.dot(agg_ref[...], w2a_ref[...],
                     preferred_element_type=jnp.float32))
    hs2_ref[...] = dinv * h2l


def _finish_body(accp_ref, hs2_ref, dinv_ref, b2_ref, out_ref):
    pre = accp_ref[0, 0] + accp_ref[1, 0] + hs2_ref[...]
    out_ref[...] = dinv_ref[...] * pre + b2_ref[...]


def _rows(i):
    return (i, 0)


def _full(i):
    return (0, 0)


def _acc_map(i):
    return (0, i // _HB, i % _HB, 0)


_spec_nd = pl.BlockSpec((_BLK, D), _rows)
_spec_nde = pl.BlockSpec((_BLK, DE), _rows)
_spec_n1 = pl.BlockSpec((_BLK, 1), _rows)
_spec_aggp = pl.BlockSpec((NC, 1, _BLK, DE), _acc_map)
_spec_accp = pl.BlockSpec((NC, 1, _BLK, D), _acc_map)
_spec_wx = pl.BlockSpec((D, D), _full)
_spec_wa = pl.BlockSpec((DE, D), _full)
_spec_b = pl.BlockSpec((1, D), _full)

_dense1 = pl.pallas_call(
    _dense1_body,
    grid=(_GRID,),
    in_specs=[_spec_nd, _spec_aggp, _spec_aggp, _spec_wx, _spec_wa],
    out_specs=[_spec_nd, _spec_n1, _spec_nde],
    out_shape=[
        jax.ShapeDtypeStruct((NPAD, D), jnp.float32),
        jax.ShapeDtypeStruct((NPAD, 1), jnp.float32),
        jax.ShapeDtypeStruct((NPAD, DE), jnp.float32),
    ],
)

_dense2 = pl.pallas_call(
    _dense2_body,
    grid=(_GRID,),
    in_specs=[_spec_accp, _spec_nd, _spec_n1, _spec_nde, _spec_wx,
              _spec_wa, _spec_b],
    out_specs=_spec_nd,
    out_shape=jax.ShapeDtypeStruct((NPAD, D), jnp.float32),
)

_finish = pl.pallas_call(
    _finish_body,
    grid=(_GRID,),
    in_specs=[_spec_accp, _spec_nd, _spec_n1, _spec_b],
    out_specs=_spec_nd,
    out_shape=jax.ShapeDtypeStruct((NPAD, D), jnp.float32),
)


def kernel(x, edge_index, edge_attr, W1, b1, W2, b2):
    row = edge_index[0]
    col = edge_index[1]
    dummy = HN + (jnp.arange(E, dtype=jnp.int32) % 128)
    rowp = jnp.stack([
        jnp.where(row < HN, row, dummy),
        jnp.where(row >= HN, row - HN, dummy),
    ]).reshape(2, NW, NCH, C)
    colp = jnp.stack([
        jnp.where(col < HN, col, dummy),
        jnp.where(col >= HN, col - HN, dummy),
    ]).reshape(2, NW, NCH, C)
    row3 = row.reshape(NW, NCH, C)
    x_pad = jnp.pad(x, ((0, NPAD - N), (0, 0)))
    e0 = jnp.concatenate(
        [jnp.ones((C, 1), jnp.float32), jnp.zeros((C, DE - 1), jnp.float32)],
        axis=1)
    z16 = jnp.zeros((RPT, DE), jnp.float32)
    z128 = jnp.zeros((ZR, D), jnp.float32)
    W1x, W1a = W1[:D], W1[D:]
    W2x, W2a = W2[:D], W2[D:]

    agg_p, deg_p = _edge_agg_deg(rowp, colp, edge_attr, e0, z16)
    hs1, dinv, agg = _dense1(x_pad, agg_p, deg_p, W1x, W1a)
    acc1 = _spmm(hs1, row3, colp, z128)
    hs2 = _dense2(acc1, hs1, dinv, agg, W2x, W2a, b1.reshape(1, D))
    acc2 = _spmm(hs2, row3, colp, z128)
    out = _finish(acc2, hs2, dinv, b2.reshape(1, D))
    return out[:N]


# speedup vs baseline: 6.3829x; 1.0104x over previous
import functools

import jax
import jax.numpy as jnp
from jax import lax
from jax.experimental import pallas as pl
from jax.experimental.pallas import tpu as pltpu
from jax.experimental.pallas import tpu_sc as plsc

N = 10000
E = 320000
D = 128
DE = 16
NPAD = 10240
NC, NS = 2, 16
NW = NC * NS
EPR = E // NW
C = 80
EPW = 10000
PADE = EPW - EPR
NCH = EPW // C
HN = NPAD // 2
ACCH = 6144
RPT = ACCH // NS
ZR = RPT // 4

_mesh = plsc.VectorSubcoreMesh(core_axis_name="c", subcore_axis_name="s")


@functools.partial(
    pl.kernel,
    out_type=(
        jax.ShapeDtypeStruct((NC, 2, ACCH, DE), jnp.float32),
        jax.ShapeDtypeStruct((NC, 2, ACCH, DE), jnp.float32),
    ),
    mesh=_mesh,
    scratch_types=[
        pltpu.VMEM((C,), jnp.int32),
        pltpu.VMEM((C,), jnp.int32),
        pltpu.VMEM((C,), jnp.int32),
        pltpu.VMEM((C,), jnp.int32),
        pltpu.VMEM((C, DE), jnp.float32),
        pltpu.VMEM((C, DE), jnp.float32),
        pltpu.VMEM((C, DE), jnp.float32),
        pltpu.VMEM((RPT, DE), jnp.float32),
        pltpu.VMEM_SHARED((ACCH, DE), jnp.float32),
        pltpu.VMEM_SHARED((ACCH, DE), jnp.float32),
        pltpu.SemaphoreType.DMA,
        pltpu.SemaphoreType.DMA,
        pltpu.SemaphoreType.DMA,
        pltpu.SemaphoreType.DMA,
        pltpu.SemaphoreType.DMA,
        pltpu.SemaphoreType.DMA,
    ],
)
def _edge_agg_deg(rowp, colp, attr, e0, z16, agg_out, deg_out,
                  ridx0, ridx1, cidx0, cidx1, abuf0, abuf1, e0_v, z_v,
                  agg_s, deg_s, sa0, sa1, ssa0, ssa1, ssd0, ssd1):
    c = lax.axis_index("c")
    s = lax.axis_index("s")
    wid = c * NS + s
    ridx = (ridx0, ridx1)
    cidx = (cidx0, cidx1)
    abuf = (abuf0, abuf1)
    sa = (sa0, sa1)
    ssa = (ssa0, ssa1)
    ssd = (ssd0, ssd1)
    pltpu.sync_copy(e0, e0_v)
    pltpu.sync_copy(z16, z_v)

    for p in range(2):
        pltpu.sync_copy(z_v, agg_s.at[pl.ds(s * RPT, RPT)])
        pltpu.sync_copy(z_v, deg_s.at[pl.ds(s * RPT, RPT)])
        plsc.subcore_barrier()

        def body(j, carry):
            pltpu.sync_copy(rowp.at[p, wid, j], ridx0)
            pltpu.sync_copy(colp.at[p, wid, j], cidx0)
            pltpu.sync_copy(attr.at[pl.ds(wid * EPW + j * C, C)], abuf0)
            pltpu.sync_copy(abuf0, agg_s.at[ridx0], add=True)
            pltpu.sync_copy(e0_v, deg_s.at[cidx0], add=True)
            return carry

        lax.fori_loop(0, NCH, body, 0)
        plsc.subcore_barrier()
        pltpu.sync_copy(agg_s.at[pl.ds(s * RPT, RPT)],
                        agg_out.at[c, p, pl.ds(s * RPT, RPT)])
        pltpu.sync_copy(deg_s.at[pl.ds(s * RPT, RPT)],
                        deg_out.at[c, p, pl.ds(s * RPT, RPT)])


@functools.partial(
    pl.kernel,
    out_type=jax.ShapeDtypeStruct((NC, 2, ACCH, D), jnp.float32),
    mesh=_mesh,
    scratch_types=[
        pltpu.VMEM((C,), jnp.int32),
        pltpu.VMEM((C,), jnp.int32),
        pltpu.VMEM((C,), jnp.int32),
        pltpu.VMEM((C,), jnp.int32),
        pltpu.VMEM((C, D), jnp.float32),
        pltpu.VMEM((C, D), jnp.float32),
        pltpu.VMEM((ZR, D), jnp.float32),
        pltpu.VMEM_SHARED((ACCH, D), jnp.float32),
        pltpu.SemaphoreType.DMA,
        pltpu.SemaphoreType.DMA,
        pltpu.SemaphoreType.DMA,
        pltpu.SemaphoreType.DMA,
    ],
)
def _spmm(hs, row3, colp, z128, out, ridx0, ridx1, cidx0, cidx1,
          gbuf0, gbuf1, zbuf, acc_s, sg0, sg1, ss0, ss1):
    c = lax.axis_index("c")
    s = lax.axis_index("s")
    wid = c * NS + s
    ridx = (ridx0, ridx1)
    cidx = (cidx0, cidx1)
    gbuf = (gbuf0, gbuf1)
    sg = (sg0, sg1)
    ss = (ss0, ss1)
    pltpu.sync_copy(z128, zbuf)
    for p in range(2):
        for k in range(4):
            pltpu.sync_copy(zbuf, acc_s.at[pl.ds(s * RPT + k * ZR, ZR)])
        plsc.subcore_barrier()

        for b in range(2):
            pltpu.sync_copy(row3.at[wid, b], ridx[b])
            pltpu.sync_copy(colp.at[p, wid, b], cidx[b])
            pltpu.async_copy(hs.at[ridx[b]], gbuf[b], sg[b])

        def slot(b, jj):
            pltpu.make_async_copy(hs.at[ridx[b]], gbuf[b], sg[b]).wait()
            pltpu.async_copy(gbuf[b], acc_s.at[cidx[b]], ss[b], add=True)

            @pl.when(jj + 2 < NCH)
            def _():
                pltpu.make_async_copy(gbuf[b], acc_s.at[cidx[b]],
                                      ss[b]).wait()
                pltpu.sync_copy(row3.at[wid, jj + 2], ridx[b])
                pltpu.sync_copy(colp.at[p, wid, jj + 2], cidx[b])
                pltpu.async_copy(hs.at[ridx[b]], gbuf[b], sg[b])

        def body(i, carry):
            slot(0, 2 * i)

            @pl.when(2 * i + 1 < NCH)
            def _():
                slot(1, 2 * i + 1)

            return carry

        lax.fori_loop(0, (NCH + 1) // 2, body, 0)
        for b in range(2):
            pltpu.make_async_copy(gbuf[b], acc_s.at[cidx[b]], ss[b]).wait()
        plsc.subcore_barrier()
        pltpu.sync_copy(acc_s.at[pl.ds(s * RPT, RPT)],
                        out.at[c, p, pl.ds(s * RPT, RPT)])


_BLK = 1024
_GRID = NPAD // _BLK
_HB = HN // _BLK


def _dense1_body(x_ref, aggp_ref, degp_ref, w1x_ref, w1a_ref,
                 hs_ref, dinv_ref, agg_ref):
    agg = aggp_ref[0, 0] + aggp_ref[1, 0]
    deg = degp_ref[0, 0, :, 0:1] + degp_ref[1, 0, :, 0:1] + 1.0
    dinv = lax.rsqrt(deg)
    hl = (jnp.dot(x_ref[...], w1x_ref[...], preferred_element_type=jnp.float32)
          + jnp.dot(agg, w1a_ref[...], preferred_element_type=jnp.float32))
    hs_ref[...] = dinv * hl
    dinv_ref[...] = dinv
    agg_ref[...] = agg


def _dense2_body(accp_ref, hs1_ref, dinv_ref, agg_ref, w2x_ref, w2a_ref,
                 b1_ref, hs2_ref):
    dinv = dinv_ref[...]
    pre = accp_ref[0, 0] + accp_ref[1, 0] + hs1_ref[...]
    h1 = jnp.maximum(dinv * pre + b1_ref[...], 0.0)
    h2l = (jnp.dot(h1, w2x_ref[...], preferred_element_type=jnp.float32)
           + jnp.dot(agg_ref[...], w2a_ref[...],
                     preferred_element_type=jnp.float32))
    hs2_ref[...] = dinv * h2l


def _finish_body(accp_ref, hs2_ref, dinv_ref, b2_ref, out_ref):
    pre = accp_ref[0, 0] + accp_ref[1, 0] + hs2_ref[...]
    out_ref[...] = dinv_ref[...] * pre + b2_ref[...]


def _rows(i):
    return (i, 0)


def _full(i):
    return (0, 0)


def _acc_map(i):
    return (0, i // _HB, i % _HB, 0)


_spec_nd = pl.BlockSpec((_BLK, D), _rows)
_spec_nde = pl.BlockSpec((_BLK, DE), _rows)
_spec_n1 = pl.BlockSpec((_BLK, 1), _rows)
_spec_aggp = pl.BlockSpec((NC, 1, _BLK, DE), _acc_map)
_spec_accp = pl.BlockSpec((NC, 1, _BLK, D), _acc_map)
_spec_wx = pl.BlockSpec((D, D), _full)
_spec_wa = pl.BlockSpec((DE, D), _full)
_spec_b = pl.BlockSpec((1, D), _full)

_dense1 = pl.pallas_call(
    _dense1_body,
    grid=(_GRID,),
    in_specs=[_spec_nd, _spec_aggp, _spec_aggp, _spec_wx, _spec_wa],
    out_specs=[_spec_nd, _spec_n1, _spec_nde],
    out_shape=[
        jax.ShapeDtypeStruct((NPAD, D), jnp.float32),
        jax.ShapeDtypeStruct((NPAD, 1), jnp.float32),
        jax.ShapeDtypeStruct((NPAD, DE), jnp.float32),
    ],
)

_dense2 = pl.pallas_call(
    _dense2_body,
    grid=(_GRID,),
    in_specs=[_spec_accp, _spec_nd, _spec_n1, _spec_nde, _spec_wx,
              _spec_wa, _spec_b],
    out_specs=_spec_nd,
    out_shape=jax.ShapeDtypeStruct((NPAD, D), jnp.float32),
)

_finish = pl.pallas_call(
    _finish_body,
    grid=(_GRID,),
    in_specs=[_spec_accp, _spec_nd, _spec_n1, _spec_b],
    out_specs=_spec_nd,
    out_shape=jax.ShapeDtypeStruct((NPAD, D), jnp.float32),
)


def _pad_edges(a, pad_vals):
    if PADE == 0:
        return a.reshape(NW, EPW)
    return jnp.concatenate(
        [a.reshape(NW, EPR),
         jnp.broadcast_to(pad_vals[None, :], (NW, PADE))], axis=1)


def kernel(x, edge_index, edge_attr, W1, b1, W2, b2):
    row = edge_index[0]
    col = edge_index[1]
    dummy = HN + (jnp.arange(E, dtype=jnp.int32) % 128)
    pad_dummy = HN + (jnp.arange(PADE, dtype=jnp.int32) % 128)
    pad_src = (jnp.arange(PADE, dtype=jnp.int32) * 37) % N
    rowp = jnp.stack([
        _pad_edges(jnp.where(row < HN, row, dummy), pad_dummy),
        _pad_edges(jnp.where(row >= HN, row - HN, dummy), pad_dummy),
    ]).reshape(2, NW, NCH, C)
    colp = jnp.stack([
        _pad_edges(jnp.where(col < HN, col, dummy), pad_dummy),
        _pad_edges(jnp.where(col >= HN, col - HN, dummy), pad_dummy),
    ]).reshape(2, NW, NCH, C)
    row3 = _pad_edges(row, pad_src).reshape(NW, NCH, C)
    if PADE == 0:
        attr_pad = edge_attr
    else:
        attr_pad = jnp.concatenate(
            [edge_attr.reshape(NW, EPR, DE),
             jnp.zeros((NW, PADE, DE), jnp.float32)],
            axis=1).reshape(NW * EPW, DE)
    x_pad = jnp.pad(x, ((0, NPAD - N), (0, 0)))
    e0 = jnp.concatenate(
        [jnp.ones((C, 1), jnp.float32), jnp.zeros((C, DE - 1), jnp.float32)],
        axis=1)
    z16 = jnp.zeros((RPT, DE), jnp.float32)
    z128 = jnp.zeros((ZR, D), jnp.float32)
    W1x, W1a = W1[:D], W1[D:]
    W2x, W2a = W2[:D], W2[D:]

    agg_p, deg_p = _edge_agg_deg(rowp, colp, attr_pad, e0, z16)
    hs1, dinv, agg = _dense1(x_pad, agg_p, deg_p, W1x, W1a)
    acc1 = _spmm(hs1, row3, colp, z128)
    hs2 = _dense2(acc1, hs1, dinv, agg, W2x, W2a, b1.reshape(1, D))
    acc2 = _spmm(hs2, row3, colp, z128)
    out = _finish(acc2, hs2, dinv, b2.reshape(1, D))
    return out[:N]
